# baseline (speedup 1.0000x reference)
import jax
import jax.numpy as jnp
from jax import lax
from jax.experimental import pallas as pl
from jax.experimental.pallas import tpu as pltpu
from jax.experimental.pallas import tpu_sc as plsc

_B = 1024
_V = 100000
_NC = 2
_NS = 16
_NW = _NC * _NS
_BPW = _B // _NW
_L = 16


def _gather_body(logits_hbm, target_hbm, part_hbm, idx_v, vals_v, part_v, sem):
    wid = lax.axis_index("s") * _NC + lax.axis_index("c")
    base = wid * _BPW

    pltpu.sync_copy(target_hbm.at[pl.ds(base, _BPW)], idx_v)

    lane = lax.iota(jnp.int32, _L)
    for j in range(_BPW // _L):
        t16 = idx_v[pl.ds(j * _L, _L)]
        rows = (base + j * _L) + lane
        idx_v[pl.ds(j * _L, _L)] = t16 + rows * _V

    pltpu.async_copy(logits_hbm.at[idx_v], vals_v, sem).wait()

    p = vals_v[pl.ds(0, _L)]
    for j in range(1, _BPW // _L):
        p = p + vals_v[pl.ds(j * _L, _L)]
    part_v[...] = p
    pltpu.sync_copy(part_v, part_hbm.at[wid])


def _reduce_body(part_hbm, out_hbm, acc_v, res_v):
    wid = lax.axis_index("s")

    @pl.when(wid == 0)
    def _():
        pltpu.sync_copy(part_hbm, acc_v)
        tot = acc_v[0]
        for j in range(1, _NW):
            tot = tot + acc_v[j]
        s = tot[0]
        for j in range(1, _L):
            s = s + tot[j]
        res_v[...] = jnp.broadcast_to(-s, (_L,))
        pltpu.sync_copy(res_v, out_hbm)


def kernel(logits, target):
    flat = logits.reshape(-1)
    tgt = target.astype(jnp.int32)

    gather_fn = pl.kernel(
        _gather_body,
        mesh=plsc.VectorSubcoreMesh(core_axis_name="c", subcore_axis_name="s"),
        out_type=jax.ShapeDtypeStruct((_NW, _L), jnp.float32),
        scratch_types=[
            pltpu.VMEM((_BPW,), jnp.int32),
            pltpu.VMEM((_BPW,), jnp.float32),
            pltpu.VMEM((_L,), jnp.float32),
            pltpu.SemaphoreType.DMA,
        ],
    )
    reduce_fn = pl.kernel(
        _reduce_body,
        mesh=plsc.VectorSubcoreMesh(
            core_axis_name="c", subcore_axis_name="s", num_cores=1),
        out_type=jax.ShapeDtypeStruct((_L,), jnp.float32),
        scratch_types=[
            pltpu.VMEM((_NW, _L), jnp.float32),
            pltpu.VMEM((_L,), jnp.float32),
        ],
    )
    parts = gather_fn(flat, tgt)
    out = reduce_fn(parts)
    return out[0]


# speedup vs baseline: 35.3325x; 35.3325x over previous
import jax
import jax.numpy as jnp
from jax import lax
from jax.experimental import pallas as pl
from jax.experimental.pallas import tpu as pltpu
from jax.experimental.pallas import tpu_sc as plsc

_B = 1024
_V = 100000
_NC = 2
_NS = 16
_NW = _NC * _NS
_BPW = _B // _NW
_L = 16


def _gather_body(logits_hbm, target_hbm, part_hbm, idx_v, vals_v, part_v, sem):
    wid = lax.axis_index("s") * _NC + lax.axis_index("c")
    base = wid * _BPW

    pltpu.sync_copy(target_hbm.at[pl.ds(base, _BPW)], idx_v)

    lane = lax.iota(jnp.int32, _L)
    for j in range(_BPW // _L):
        t16 = idx_v[pl.ds(j * _L, _L)]
        rows = (base + j * _L) + lane
        addr = (
            ((t16 >> 3) << 13)
            + ((rows >> 7) << 10)
            + ((t16 & 7) << 7)
            + (rows & 127)
        )
        idx_v[pl.ds(j * _L, _L)] = addr

    pltpu.async_copy(logits_hbm.at[idx_v], vals_v, sem).wait()

    p = vals_v[pl.ds(0, _L)]
    for j in range(1, _BPW // _L):
        p = p + vals_v[pl.ds(j * _L, _L)]
    part_v[...] = p
    pltpu.sync_copy(part_v, part_hbm.at[wid])


def _reduce_body(part_hbm, out_hbm, acc_v, res_v):
    wid = lax.axis_index("s")

    @pl.when(wid == 0)
    def _():
        pltpu.sync_copy(part_hbm, acc_v)
        tot = acc_v[0]
        for j in range(1, _NW):
            tot = tot + acc_v[j]
        s = tot[0]
        for j in range(1, _L):
            s = s + tot[j]
        res_v[...] = jnp.broadcast_to(-s, (_L,))
        pltpu.sync_copy(res_v, out_hbm)


def kernel(logits, target):
    flat = (
        logits.reshape(8, 128, 12500, 8).transpose(2, 0, 3, 1).reshape(-1)
    )
    tgt = target.astype(jnp.int32)

    gather_fn = pl.kernel(
        _gather_body,
        mesh=plsc.VectorSubcoreMesh(core_axis_name="c", subcore_axis_name="s"),
        out_type=jax.ShapeDtypeStruct((_NW, _L), jnp.float32),
        scratch_types=[
            pltpu.VMEM((_BPW,), jnp.int32),
            pltpu.VMEM((_BPW,), jnp.float32),
            pltpu.VMEM((_L,), jnp.float32),
            pltpu.SemaphoreType.DMA,
        ],
    )
    reduce_fn = pl.kernel(
        _reduce_body,
        mesh=plsc.VectorSubcoreMesh(
            core_axis_name="c", subcore_axis_name="s", num_cores=1),
        out_type=jax.ShapeDtypeStruct((_L,), jnp.float32),
        scratch_types=[
            pltpu.VMEM((_NW, _L), jnp.float32),
            pltpu.VMEM((_L,), jnp.float32),
        ],
    )
    parts = gather_fn(flat, tgt)
    out = reduce_fn(parts)
    return out[0]


# speedup vs baseline: 42.0677x; 1.1906x over previous
import jax
import jax.numpy as jnp
from jax import lax
from jax.experimental import pallas as pl
from jax.experimental.pallas import tpu as pltpu
from jax.experimental.pallas import tpu_sc as plsc

_B = 1024
_L = 16
_CHUNK = 128
_NCHUNKS = _B // _CHUNK


def _sc_body(logits_hbm, target_hbm, out_hbm, idx_v, vals_v, sem):
    wid = lax.axis_index("s")

    @pl.when(wid == 0)
    def _():
        pltpu.sync_copy(target_hbm, idx_v)

        lane = lax.iota(jnp.int32, _L)
        for j in range(_B // _L):
            t16 = idx_v[pl.ds(j * _L, _L)]
            rows = (j * _L) + lane
            addr = (
                ((t16 >> 3) << 13)
                + ((rows >> 7) << 10)
                + ((t16 & 7) << 7)
                + (rows & 127)
            )
            idx_v[pl.ds(j * _L, _L)] = addr

        copies = []
        for k in range(_NCHUNKS):
            copies.append(pltpu.async_copy(
                logits_hbm.at[idx_v.at[pl.ds(k * _CHUNK, _CHUNK)]],
                vals_v.at[pl.ds(k * _CHUNK, _CHUNK)],
                sem,
            ))
        for c in copies:
            c.wait()

        p = vals_v[pl.ds(0, _L)]
        for j in range(1, _B // _L):
            p = p + vals_v[pl.ds(j * _L, _L)]
        s = p[0]
        for j in range(1, _L):
            s = s + p[j]
        vals_v[pl.ds(0, _L)] = jnp.broadcast_to(-s, (_L,))
        pltpu.sync_copy(vals_v.at[pl.ds(0, _L)], out_hbm)


def kernel(logits, target):
    flat = (
        logits.reshape(8, 128, 12500, 8).transpose(2, 0, 3, 1).reshape(-1)
    )
    tgt = target.astype(jnp.int32)

    fn = pl.kernel(
        _sc_body,
        mesh=plsc.VectorSubcoreMesh(
            core_axis_name="c", subcore_axis_name="s", num_cores=1),
        out_type=jax.ShapeDtypeStruct((_L,), jnp.float32),
        scratch_types=[
            pltpu.VMEM((_B,), jnp.int32),
            pltpu.VMEM((_B,), jnp.float32),
            pltpu.SemaphoreType.DMA,
        ],
    )
    out = fn(flat, tgt)
    return out[0]
